# baseline (speedup 1.0000x reference)
import functools

import jax
import jax.numpy as jnp
from jax import lax
from jax.experimental import pallas as pl
from jax.experimental.pallas import tpu as pltpu
from jax.experimental.pallas import tpu_sc as plsc

EMB_DIM = 64
LANE = 128
CR = 4
CHUNK = CR * LANE


def kernel(data, table):
    B = data.shape[0] * data.shape[1]
    idx2d = data.reshape(B // LANE, LANE)

    info = plsc.get_sparse_core_info()
    nw = info.num_cores * info.num_subcores
    nr_per_w = (B // LANE) // nw
    n_chunks = nr_per_w // CR

    mesh = plsc.VectorSubcoreMesh(core_axis_name="c", subcore_axis_name="s")

    @functools.partial(
        pl.kernel,
        mesh=mesh,
        out_type=jax.ShapeDtypeStruct((B, EMB_DIM), jnp.float32),
        scratch_types=[
            pltpu.VMEM((CR, LANE), jnp.int32),
            pltpu.VMEM((CHUNK, EMB_DIM), jnp.float32),
            pltpu.SemaphoreType.DMA,
        ],
        compiler_params=pltpu.CompilerParams(use_tc_tiling_on_sc=False),
    )
    def run(idx_hbm, table_hbm, out_hbm, idx_v, rows_v, sem):
        wid = lax.axis_index("s") * info.num_cores + lax.axis_index("c")
        row0 = wid * nr_per_w

        def body(g, carry):
            r = row0 + g * CR
            pltpu.sync_copy(idx_hbm.at[pl.ds(r, CR)], idx_v)
            copies = [
                pltpu.async_copy(
                    table_hbm.at[idx_v.at[j]],
                    rows_v.at[pl.ds(j * LANE, LANE)],
                    sem,
                )
                for j in range(CR)
            ]
            for cp in copies:
                cp.wait()
            pltpu.sync_copy(rows_v, out_hbm.at[pl.ds(r * LANE, CHUNK)])
            return carry

        lax.fori_loop(0, n_chunks, body, 0)

    out = run(idx2d, table)
    return out.reshape(data.shape[0], data.shape[1], EMB_DIM)


# speedup vs baseline: 1.0444x; 1.0444x over previous
import functools

import jax
import jax.numpy as jnp
from jax import lax
from jax.experimental import pallas as pl
from jax.experimental.pallas import tpu as pltpu
from jax.experimental.pallas import tpu_sc as plsc

EMB_DIM = 64
LANE = 128
CR = 5
CHUNK = CR * LANE
N_BUF = 2


def kernel(data, table):
    B = data.shape[0] * data.shape[1]
    idx2d = data.reshape(B // LANE, LANE)

    info = plsc.get_sparse_core_info()
    nw = info.num_cores * info.num_subcores
    nr_per_w = (B // LANE) // nw
    n_chunks = nr_per_w // CR

    mesh = plsc.VectorSubcoreMesh(core_axis_name="c", subcore_axis_name="s")

    @functools.partial(
        pl.kernel,
        mesh=mesh,
        out_type=jax.ShapeDtypeStruct((B, EMB_DIM), jnp.float32),
        scratch_types=[
            pltpu.VMEM((nr_per_w, LANE), jnp.int32),
            pltpu.VMEM((N_BUF, CHUNK, EMB_DIM), jnp.float32),
            pltpu.SemaphoreType.DMA((N_BUF,)),
            pltpu.SemaphoreType.DMA((N_BUF,)),
        ],
        compiler_params=pltpu.CompilerParams(use_tc_tiling_on_sc=False),
    )
    def run(idx_hbm, table_hbm, out_hbm, idx_all, rows_v, gsem, ssem):
        wid = lax.axis_index("s") * info.num_cores + lax.axis_index("c")
        row0 = wid * nr_per_w
        pltpu.sync_copy(idx_hbm.at[pl.ds(row0, nr_per_w)], idx_all)

        def fire_gathers(g, b):
            for j in range(CR):
                pltpu.async_copy(
                    table_hbm.at[idx_all.at[g * CR + j]],
                    rows_v.at[b].at[pl.ds(j * LANE, LANE)],
                    gsem.at[b],
                )

        def wait_gathers(b):
            for _ in range(CR):
                pltpu.make_async_copy(
                    table_hbm.at[idx_all.at[0]],
                    rows_v.at[b].at[pl.ds(0, LANE)],
                    gsem.at[b],
                ).wait()

        def start_store(g, b):
            r = (row0 + g * CR) * LANE
            pltpu.async_copy(rows_v.at[b], out_hbm.at[pl.ds(r, CHUNK)], ssem.at[b])

        def wait_store(b):
            pltpu.make_async_copy(
                rows_v.at[b], out_hbm.at[pl.ds(0, CHUNK)], ssem.at[b]
            ).wait()

        for b in range(N_BUF):
            fire_gathers(b, b)

        def body(i, carry):
            for b in range(N_BUF):
                g = i * N_BUF + b
                wait_gathers(b)
                start_store(g, b)
                nxt = g + N_BUF

                @pl.when(nxt < n_chunks)
                def _():
                    wait_store(b)
                    fire_gathers(nxt, b)

            return carry

        lax.fori_loop(0, n_chunks // N_BUF, body, 0)
        for b in range(N_BUF):
            wait_store(b)

    out = run(idx2d, table)
    return out.reshape(data.shape[0], data.shape[1], EMB_DIM)
